# baseline (speedup 1.0000x reference)
import functools

import jax
import jax.numpy as jnp
from jax import lax
from jax.experimental import pallas as pl
from jax.experimental.pallas import tpu as pltpu
from jax.experimental.pallas import tpu_sc as plsc

_B = 16
_SRC = 200
_NT = 16
_NFULL = _SRC // 16
_REM = _SRC - _NFULL * 16
_TAIL = _SRC - 16
_MTAIL = _SRC - 64
_LN2 = 0.6931471805599453

_mesh = plsc.VectorSubcoreMesh(core_axis_name="c", subcore_axis_name="s",
                               num_cores=1)


def _lanesum(v, lane):
    for s in (8, 4, 2, 1):
        v = v + v.at[lane ^ s].get(mode="promise_in_bounds",
                                   unique_indices=True)
    return v


def _mask_words(mv, woff):
    return mv[pl.ds(woff, 16)]


def _keep(words, q, lq, shamt):
    w = words.at[4 * q + lq].get(mode="promise_in_bounds")
    m = (w >> shamt) & 1
    return 1.0 - m.astype(jnp.float32)


@functools.partial(
    pl.kernel,
    mesh=_mesh,
    out_type=[jax.ShapeDtypeStruct((1,), jnp.float32)] * 6,
    compiler_params=pltpu.CompilerParams(needs_layout_passes=False,
                                         skip_device_barrier=True),
    scratch_types=[
        pltpu.VMEM((_SRC,), jnp.float32),
        pltpu.VMEM((_SRC,), jnp.float32),
        pltpu.VMEM((_SRC,), jnp.float32),
        pltpu.VMEM((_SRC,), jnp.float32),
        pltpu.VMEM((_SRC,), jnp.float32),
        pltpu.VMEM((_SRC,), jnp.float32),
        pltpu.VMEM((_SRC // 4,), jnp.int32),
        pltpu.VMEM((64,), jnp.float32),
        pltpu.VMEM_SHARED((_NT * 64,), jnp.float32),
        pltpu.VMEM((_NT * 64,), jnp.float32),
        pltpu.VMEM((16,), jnp.float32),
        pltpu.VMEM((16,), jnp.int32),
        pltpu.VMEM((16,), jnp.float32),
        pltpu.VMEM((96,), jnp.float32),
        pltpu.SemaphoreType.DMA,
    ],
)
def _sc_loss(pp, pt, ep, et, dp, dt, sm, mlp, mlt, pg,
             o_tot, o_pit, o_ene, o_dur, o_fd, o_g,
             ppv, ptv, epv, etv, dpv, dtv, smv,
             accv, shared, redv, mlpv, mltv, pgv, outv, sem):
    sid = lax.axis_index("s")

    cps = [
        pltpu.async_copy(pp.at[sid], ppv, sem),
        pltpu.async_copy(pt.at[sid], ptv, sem),
        pltpu.async_copy(ep.at[sid], epv, sem),
        pltpu.async_copy(et.at[sid], etv, sem),
        pltpu.async_copy(dp.at[sid], dpv, sem),
        pltpu.async_copy(dt.at[sid], dtv, sem),
        pltpu.async_copy(sm.at[sid], smv, sem),
        pltpu.async_copy(mlp, mlpv, sem),
        pltpu.async_copy(mlt, mltv, sem),
        pltpu.async_copy(pg, pgv, sem),
    ]
    for c in cps:
        c.wait()

    lane = lax.broadcasted_iota(jnp.int32, (16,), 0)
    lq = lane >> 2
    shamt = (lane & 3) * 8
    accp = jnp.zeros((16,), jnp.float32)
    acce = jnp.zeros((16,), jnp.float32)
    accd = jnp.zeros((16,), jnp.float32)
    accc = jnp.zeros((16,), jnp.float32)
    wblocks = [_mask_words(smv, 16 * b) for b in range(_NFULL // 4)]
    wtail = _mask_words(smv, _MTAIL // 4)
    for j in range(_NFULL + (1 if _REM else 0)):
        if j < _NFULL:
            off = j * 16
            m = _keep(wblocks[j // 4], j % 4, lq, shamt)
        else:
            off = _TAIL
            m = _keep(wtail, (_TAIL - _MTAIL) // 16, lq, shamt)
            m = jnp.where(lane >= 16 - _REM, m, 0.0)
        d0 = ppv[pl.ds(off, 16)] - ptv[pl.ds(off, 16)]
        d1 = epv[pl.ds(off, 16)] - etv[pl.ds(off, 16)]
        d2 = dpv[pl.ds(off, 16)] - dtv[pl.ds(off, 16)]
        accp = accp + d0 * d0 * m
        acce = acce + d1 * d1 * m
        accd = accd + d2 * d2 * m
        accc = accc + m

    accv[pl.ds(0, 16)] = accp
    accv[pl.ds(16, 16)] = acce
    accv[pl.ds(32, 16)] = accd
    accv[pl.ds(48, 16)] = accc
    pltpu.sync_copy(accv, shared.at[pl.ds(sid * 64, 64)])
    plsc.subcore_barrier()

    @pl.when(sid == 0)
    def _tile0():
        pltpu.sync_copy(shared, redv)
        sp = jnp.zeros((16,), jnp.float32)
        se = jnp.zeros((16,), jnp.float32)
        sd = jnp.zeros((16,), jnp.float32)
        sc = jnp.zeros((16,), jnp.float32)
        for t in range(_NT):
            sp = sp + redv[pl.ds(t * 64, 16)]
            se = se + redv[pl.ds(t * 64 + 16, 16)]
            sd = sd + redv[pl.ds(t * 64 + 32, 16)]
            sc = sc + redv[pl.ds(t * 64 + 48, 16)]
        inv = 1.0 / jnp.maximum(_lanesum(sc, lane), 1.0)
        pitch = 0.5 * _lanesum(sp, lane) * inv
        energy = 0.5 * _lanesum(se, lane) * inv
        duration = _lanesum(sd, lane) * inv

        mltf = mltv[...].astype(jnp.float32)
        fd = _lanesum(jnp.abs(mlpv[...] - mltf), lane) * (0.01 / 16.0)

        q = pgv[...]
        bits = plsc.bitcast(q, jnp.int32)
        e = (bits >> 23) - 127
        mant = plsc.bitcast((bits & 0x007FFFFF) | 0x3F800000, jnp.float32)
        r = (mant - 1.0) / (mant + 1.0)
        r2 = r * r
        lgm = 2.0 * r * (1.0 + r2 * (1.0 / 3.0 + r2 * (0.2 + r2 * (1.0 / 7.0))))
        lg = e.astype(jnp.float32) * _LN2 + lgm
        g = _lanesum(-jnp.maximum(lg, -100.0), lane) * (1.0 / 16.0)

        total = pitch + energy + duration + fd + g
        outv[pl.ds(0, 16)] = total
        outv[pl.ds(16, 16)] = pitch
        outv[pl.ds(32, 16)] = energy
        outv[pl.ds(48, 16)] = duration
        outv[pl.ds(64, 16)] = fd
        outv[pl.ds(80, 16)] = g
        ocs = [
            pltpu.async_copy(outv.at[pl.ds(0, 1)], o_tot, sem),
            pltpu.async_copy(outv.at[pl.ds(16, 1)], o_pit, sem),
            pltpu.async_copy(outv.at[pl.ds(32, 1)], o_ene, sem),
            pltpu.async_copy(outv.at[pl.ds(48, 1)], o_dur, sem),
            pltpu.async_copy(outv.at[pl.ds(64, 1)], o_fd, sem),
            pltpu.async_copy(outv.at[pl.ds(80, 1)], o_g, sem),
        ]
        for c in ocs:
            c.wait()


def kernel(text, mel_targets, mel_lens_targets, pitch_targets,
           energy_targets, log_duration_targets, mel_predictions,
           postnet_mel_predictions, pitch_predictions, energy_predictions,
           log_duration_predictions, p_placeholder, src_masks, mel_masks,
           mel_placeholder, mel_lens_predictions, extracted_e, log_pi, mu,
           sigma, pred_generated):
    tot, pit, ene, dur, fd, g = _sc_loss(
        pitch_predictions, pitch_targets,
        energy_predictions, energy_targets,
        log_duration_predictions, log_duration_targets,
        src_masks.view(jnp.int32),
        mel_lens_predictions, mel_lens_targets,
        pred_generated.reshape(-1))
    z = jnp.zeros((), jnp.float32)
    return (tot.reshape(()), z, z, pit.reshape(()), ene.reshape(()),
            dur.reshape(()), z, z, fd.reshape(()), g.reshape(()))


# speedup vs baseline: 1.0027x; 1.0027x over previous
import functools

import jax
import jax.numpy as jnp
from jax import lax
from jax.experimental import pallas as pl
from jax.experimental.pallas import tpu as pltpu
from jax.experimental.pallas import tpu_sc as plsc

_B = 16
_SRC = 200
_NT = 16
_NFULL = _SRC // 16
_REM = _SRC - _NFULL * 16
_TAIL = _SRC - 16
_MTAIL = _SRC - 64
_LN2 = 0.6931471805599453

_mesh = plsc.VectorSubcoreMesh(core_axis_name="c", subcore_axis_name="s",
                               num_cores=1)


def _lanesum(v, lane):
    for s in (8, 4, 2, 1):
        v = v + v.at[lane ^ s].get(mode="promise_in_bounds",
                                   unique_indices=True)
    return v


def _keep_weight(mv, j, lq, shamt):
    words = mv[pl.ds((j // 4) * 16, 16)]
    w = words.at[4 * (j % 4) + lq].get(mode="promise_in_bounds")
    m = (w >> shamt) & 1
    return 1.0 - m.astype(jnp.float32)


def _keep(words, q, lq, shamt):
    w = words.at[4 * q + lq].get(mode="promise_in_bounds")
    m = (w >> shamt) & 1
    return 1.0 - m.astype(jnp.float32)


@functools.partial(
    pl.kernel,
    mesh=_mesh,
    out_type=[jax.ShapeDtypeStruct((1,), jnp.float32)] * 6,
    compiler_params=pltpu.CompilerParams(needs_layout_passes=False,
                                         skip_device_barrier=True),
    scratch_types=[
        pltpu.VMEM((_SRC,), jnp.float32),
        pltpu.VMEM((_SRC,), jnp.float32),
        pltpu.VMEM((_SRC,), jnp.float32),
        pltpu.VMEM((_SRC,), jnp.float32),
        pltpu.VMEM((_SRC,), jnp.float32),
        pltpu.VMEM((_SRC,), jnp.float32),
        pltpu.VMEM((_SRC // 4,), jnp.int32),
        pltpu.VMEM((64,), jnp.float32),
        pltpu.VMEM_SHARED((_NT * 64,), jnp.float32),
        pltpu.VMEM((_NT * 64,), jnp.float32),
        pltpu.VMEM((16,), jnp.float32),
        pltpu.VMEM((16,), jnp.int32),
        pltpu.VMEM((16,), jnp.float32),
        pltpu.VMEM((96,), jnp.float32),
        pltpu.SemaphoreType.DMA,
    ],
)
def _sc_loss(pp, pt, ep, et, dp, dt, sm, mlp, mlt, pg,
             o_tot, o_pit, o_ene, o_dur, o_fd, o_g,
             ppv, ptv, epv, etv, dpv, dtv, smv,
             accv, shared, redv, mlpv, mltv, pgv, outv, sem):
    sid = lax.axis_index("s")

    cps = [
        pltpu.async_copy(pp.at[sid], ppv, sem),
        pltpu.async_copy(pt.at[sid], ptv, sem),
        pltpu.async_copy(ep.at[sid], epv, sem),
        pltpu.async_copy(et.at[sid], etv, sem),
        pltpu.async_copy(dp.at[sid], dpv, sem),
        pltpu.async_copy(dt.at[sid], dtv, sem),
        pltpu.async_copy(sm.at[sid], smv, sem),
        pltpu.async_copy(mlp, mlpv, sem),
        pltpu.async_copy(mlt, mltv, sem),
        pltpu.async_copy(pg, pgv, sem),
    ]
    for c in cps:
        c.wait()

    lane = lax.broadcasted_iota(jnp.int32, (16,), 0)
    lq = lane >> 2
    shamt = (lane & 3) * 8
    zv = jnp.zeros((16,), jnp.float32)

    def _iter(j, off, m, carry):
        accp, acce, accd, accc = carry
        d0 = ppv[pl.ds(off, 16)] - ptv[pl.ds(off, 16)]
        d1 = epv[pl.ds(off, 16)] - etv[pl.ds(off, 16)]
        d2 = dpv[pl.ds(off, 16)] - dtv[pl.ds(off, 16)]
        return (accp + d0 * d0 * m, acce + d1 * d1 * m,
                accd + d2 * d2 * m, accc + m)

    def _body(j, carry):
        return _iter(j, j * 16, _keep_weight(smv, j, lq, shamt), carry)

    accs = lax.fori_loop(0, _NFULL, _body, (zv, zv, zv, zv))
    wt = smv[pl.ds(_SRC // 4 - 16, 16)]
    w = wt.at[(_TAIL - _MTAIL) // 4 + lq].get(mode="promise_in_bounds")
    mt = 1.0 - ((w >> shamt) & 1).astype(jnp.float32)
    mt = jnp.where(lane >= 16 - _REM, mt, 0.0)
    accp, acce, accd, accc = _iter(0, _TAIL, mt, accs)

    accv[pl.ds(0, 16)] = accp
    accv[pl.ds(16, 16)] = acce
    accv[pl.ds(32, 16)] = accd
    accv[pl.ds(48, 16)] = accc
    pltpu.sync_copy(accv, shared.at[pl.ds(sid * 64, 64)])
    plsc.subcore_barrier()

    @pl.when(sid == 0)
    def _tile0():
        pltpu.sync_copy(shared, redv)

        def _rbody(t, c):
            return (c[0] + redv[pl.ds(t * 64, 16)],
                    c[1] + redv[pl.ds(t * 64 + 16, 16)],
                    c[2] + redv[pl.ds(t * 64 + 32, 16)],
                    c[3] + redv[pl.ds(t * 64 + 48, 16)])

        sp, se, sd, sc = lax.fori_loop(0, _NT, _rbody, (zv, zv, zv, zv))
        inv = 1.0 / jnp.maximum(_lanesum(sc, lane), 1.0)
        pitch = 0.5 * _lanesum(sp, lane) * inv
        energy = 0.5 * _lanesum(se, lane) * inv
        duration = _lanesum(sd, lane) * inv

        mltf = mltv[...].astype(jnp.float32)
        fd = _lanesum(jnp.abs(mlpv[...] - mltf), lane) * (0.01 / 16.0)

        q = pgv[...]
        bits = plsc.bitcast(q, jnp.int32)
        e = (bits >> 23) - 127
        mant = plsc.bitcast((bits & 0x007FFFFF) | 0x3F800000, jnp.float32)
        r = (mant - 1.0) / (mant + 1.0)
        r2 = r * r
        lgm = 2.0 * r * (1.0 + r2 * (1.0 / 3.0 + r2 * (0.2 + r2 * (1.0 / 7.0))))
        lg = e.astype(jnp.float32) * _LN2 + lgm
        g = _lanesum(-jnp.maximum(lg, -100.0), lane) * (1.0 / 16.0)

        total = pitch + energy + duration + fd + g
        outv[pl.ds(0, 16)] = total
        outv[pl.ds(16, 16)] = pitch
        outv[pl.ds(32, 16)] = energy
        outv[pl.ds(48, 16)] = duration
        outv[pl.ds(64, 16)] = fd
        outv[pl.ds(80, 16)] = g
        ocs = [
            pltpu.async_copy(outv.at[pl.ds(0, 1)], o_tot, sem),
            pltpu.async_copy(outv.at[pl.ds(16, 1)], o_pit, sem),
            pltpu.async_copy(outv.at[pl.ds(32, 1)], o_ene, sem),
            pltpu.async_copy(outv.at[pl.ds(48, 1)], o_dur, sem),
            pltpu.async_copy(outv.at[pl.ds(64, 1)], o_fd, sem),
            pltpu.async_copy(outv.at[pl.ds(80, 1)], o_g, sem),
        ]
        for c in ocs:
            c.wait()


def kernel(text, mel_targets, mel_lens_targets, pitch_targets,
           energy_targets, log_duration_targets, mel_predictions,
           postnet_mel_predictions, pitch_predictions, energy_predictions,
           log_duration_predictions, p_placeholder, src_masks, mel_masks,
           mel_placeholder, mel_lens_predictions, extracted_e, log_pi, mu,
           sigma, pred_generated):
    tot, pit, ene, dur, fd, g = _sc_loss(
        pitch_predictions, pitch_targets,
        energy_predictions, energy_targets,
        log_duration_predictions, log_duration_targets,
        src_masks.view(jnp.int32),
        mel_lens_predictions, mel_lens_targets,
        pred_generated.reshape(-1))
    z = jnp.zeros((), jnp.float32)
    return (tot.reshape(()), z, z, pit.reshape(()), ene.reshape(()),
            dur.reshape(()), z, z, fd.reshape(()), g.reshape(()))
